# baseline (speedup 1.0000x reference)
import jax
import jax.numpy as jnp
from jax.experimental import pallas as pl
from jax.experimental.pallas import tpu as pltpu

_NCHUNK = 8


def _body(t_hbm, o_hbm, buf, sem_in, sem_out):
    B, D, P = o_hbm.shape
    rows = D // _NCHUNK
    loads = [
        pltpu.make_async_copy(
            t_hbm.at[pl.ds(i * rows, rows)],
            buf.at[pl.ds(i * rows, rows)],
            sem_in,
        )
        for i in range(_NCHUNK)
    ]
    for ld in loads:
        ld.start()
    writes = []
    for i in range(_NCHUNK):
        loads[i].wait()
        for b in range(B):
            c = pltpu.make_async_copy(
                buf.at[pl.ds(i * rows, rows)],
                o_hbm.at[b, pl.ds(i * rows, rows)],
                sem_out.at[b],
            )
            c.start()
            writes.append(c)
    for c in writes:
        c.wait()


def kernel(x, pos_table):
    B = x.shape[0]
    P, D = pos_table.shape
    table_t = pos_table.T
    out_t = pl.pallas_call(
        _body,
        in_specs=[pl.BlockSpec(memory_space=pl.ANY)],
        out_specs=pl.BlockSpec(memory_space=pl.ANY),
        out_shape=jax.ShapeDtypeStruct((B, D, P), jnp.float32),
        scratch_shapes=[
            pltpu.VMEM((D, P), jnp.float32),
            pltpu.SemaphoreType.DMA,
            pltpu.SemaphoreType.DMA((B,)),
        ],
    )(table_t)
    return jnp.transpose(out_t, (0, 2, 1))


# speedup vs baseline: 1.0077x; 1.0077x over previous
import jax
import jax.numpy as jnp
from jax.experimental import pallas as pl
from jax.experimental.pallas import tpu as pltpu

_NCHUNK = 4


def _body(t_hbm, o_hbm, buf, sem_in, sem_out):
    B, D, P = o_hbm.shape
    rows = D // _NCHUNK
    loads = [
        pltpu.make_async_copy(
            t_hbm.at[pl.ds(i * rows, rows)],
            buf.at[pl.ds(i * rows, rows)],
            sem_in,
        )
        for i in range(_NCHUNK)
    ]
    for ld in loads:
        ld.start()
    writes = []
    for i in range(_NCHUNK):
        loads[i].wait()
        for b in range(B):
            c = pltpu.make_async_copy(
                buf.at[pl.ds(i * rows, rows)],
                o_hbm.at[b, pl.ds(i * rows, rows)],
                sem_out.at[b],
            )
            c.start()
            writes.append(c)
    for c in writes:
        c.wait()


def kernel(x, pos_table):
    B = x.shape[0]
    P, D = pos_table.shape
    table_t = pos_table.T
    out_t = pl.pallas_call(
        _body,
        in_specs=[pl.BlockSpec(memory_space=pl.ANY)],
        out_specs=pl.BlockSpec(memory_space=pl.ANY),
        out_shape=jax.ShapeDtypeStruct((B, D, P), jnp.float32),
        scratch_shapes=[
            pltpu.VMEM((D, P), jnp.float32),
            pltpu.SemaphoreType.DMA,
            pltpu.SemaphoreType.DMA((B,)),
        ],
    )(table_t)
    return jnp.transpose(out_t, (0, 2, 1))
